# baseline (speedup 1.0000x reference)
import functools

import jax
import jax.numpy as jnp
from jax import lax
from jax.experimental import pallas as pl
from jax.experimental.pallas import tpu as pltpu
from jax.experimental.pallas import tpu_sc as plsc

B, A, NB, F, R = 4, 512, 32, 128, 20
E = B * A * NB
M = B * A
CUTOFF = 5.0
P = 9

NC, NS = 2, 16
NW = NC * NS
EW = E // NW
CH = 128
NCH = EW // CH
AW = M // NW


def _sigmoid(x):
    return 1.0 / (1.0 + jnp.exp(-x))


def _silu(x):
    return x * _sigmoid(x)


TB1 = 256


def _stage1_body(a_ref, qd_ref, ql_ref,
                 wa1, ba1, wa2, ba2,
                 wq1, bq1, wq2, bq2,
                 wm1, bm1, wm2, bm2,
                 we1, be1, we2, be2,
                 amsij_o, qdn_o, qln_o, emlp_o):
    x = a_ref[...]

    def mlp(w1, b1, w2, b2):
        h = _silu(jnp.dot(x, w1[...]) + b1[...])
        return jnp.dot(h, w2[...]) + b2[...]

    amsij_o[...] = mlp(wa1, ba1, wa2, ba2)
    q = mlp(wq1, bq1, wq2, bq2)
    qm = mlp(wm1, bm1, wm2, bm2)
    qdn_o[...] = qd_ref[...] + q * qm
    qln_o[...] = ql_ref[...] + q
    emlp_o[...] = mlp(we1, be1, we2, be2)


def _stage1(af, qdf, qlf, pa, pq, pqm, pe):
    n = M // TB1
    row = pl.BlockSpec((TB1, F), lambda i: (i, 0))
    col = pl.BlockSpec((TB1, 1), lambda i: (i, 0))
    wspec = lambda s: pl.BlockSpec(s, lambda i: (0, 0))
    specs_w = []
    args_w = []
    for (w1, b1, w2, b2) in (pa, pq, pqm, pe):
        args_w += [w1, b1.reshape(1, -1), w2, b2.reshape(1, -1)]
        specs_w += [wspec(w1.shape), wspec((1, b1.shape[0])),
                    wspec(w2.shape), wspec((1, b2.shape[0]))]
    return pl.pallas_call(
        _stage1_body,
        grid=(n,),
        in_specs=[row, row, col] + specs_w,
        out_specs=[row, row, col, row],
        out_shape=[
            jax.ShapeDtypeStruct((M, F), jnp.float32),
            jax.ShapeDtypeStruct((M, F), jnp.float32),
            jax.ShapeDtypeStruct((M, 1), jnp.float32),
            jax.ShapeDtypeStruct((M, F), jnp.float32),
        ],
    )(af, qdf, qlf, *args_w)


def _lane_bcast(v16, e):
    idx = jnp.full((16, 1), e, jnp.int32)
    return lax.gather(
        v16, idx,
        lax.GatherDimensionNumbers(offset_dims=(), collapsed_slice_dims=(0,),
                                   start_index_map=(0,)),
        (1,), mode=lax.GatherScatterMode.PROMISE_IN_BOUNDS)


EB = A * NB
EWB = EB // NW
NCHB = EWB // CH
AWB = A // NW


def _make_sc_kernel(b):
    roff = b * A

    def kern(nidx2, d2, amsij, qdyn, aj_o, qsum_o,
             idx_v, w_v, ab0, ab1, qb0, qb1, qs_v,
             sga0, sga1, sgq0, sgq1, swa0, swa1):
        wid = lax.axis_index("c") * NS + lax.axis_index("s")
        ebase = wid * EWB
        abase = wid * AWB

        pltpu.sync_copy(nidx2.at[pl.ds(wid * NCHB, NCHB)], idx_v)
        pltpu.sync_copy(d2.at[pl.ds(wid * NCHB, NCHB)], w_v)
        for r in range(NCHB):
            for i in range(CH // 16):
                sl = pl.ds(i * 16, 16)
                idx_v[r, sl] = idx_v[r, sl] + roff
                dd = w_v[r, sl]
                w_v[r, sl] = jnp.where(dd > 0.0, 1.0 / dd, 0.0)

        def start(c, ab, qb, sga, sgq):
            row = idx_v.at[c]
            pltpu.async_copy(amsij.at[row], ab, sga)
            pltpu.async_copy(qdyn.at[row], qb, sgq)

        start(0, ab0, qb0, sga0, sgq0)
        start(1, ab1, qb1, sga1, sgq1)

        def section(c, ab, qb, sga, sgq, swa):
            pltpu.make_async_copy(amsij.at[idx_v.at[0]], ab, sga).wait()
            pltpu.make_async_copy(qdyn.at[idx_v.at[0]], qb, sgq).wait()
            pltpu.async_copy(ab, aj_o.at[pl.ds(ebase + c * CH, CH)], swa)
            for k in range(CH // NB):
                acc = [jnp.zeros((16,), jnp.float32) for _ in range(F // 16)]
                for g in range(NB // 16):
                    w16 = w_v[c, pl.ds((k * 2 + g) * 16, 16)]
                    for e in range(16):
                        we = _lane_bcast(w16, e)
                        r = k * NB + g * 16 + e
                        for f in range(F // 16):
                            acc[f] = acc[f] + qb[r, pl.ds(f * 16, 16)] * we
                for f in range(F // 16):
                    qs_v[c * (CH // NB) + k, pl.ds(f * 16, 16)] = acc[f]
            @pl.when(c + 2 < NCHB)
            def _():
                pltpu.make_async_copy(ab, aj_o.at[pl.ds(ebase + c * CH, CH)],
                                      swa).wait()
                start(c + 2, ab, qb, sga, sgq)

        def body(co, carry):
            section(2 * co, ab0, qb0, sga0, sgq0, swa0)
            section(2 * co + 1, ab1, qb1, sga1, sgq1, swa1)
            return carry

        lax.fori_loop(0, NCHB // 2, body, 0)
        pltpu.make_async_copy(ab0, aj_o.at[pl.ds(ebase + (NCHB - 2) * CH, CH)],
                              swa0).wait()
        pltpu.make_async_copy(ab1, aj_o.at[pl.ds(ebase + (NCHB - 1) * CH, CH)],
                              swa1).wait()
        pltpu.sync_copy(qs_v, qsum_o.at[pl.ds(abase, AWB)])

    return kern


@functools.cache
def _sc_gather_built(b):
    return functools.partial(
        pl.kernel,
        mesh=plsc.VectorSubcoreMesh(core_axis_name="c", subcore_axis_name="s"),
        out_type=[
            jax.ShapeDtypeStruct((EB, F), jnp.float32),
            jax.ShapeDtypeStruct((A, F), jnp.float32),
        ],
        scratch_types=[
            pltpu.VMEM((NCHB, CH), jnp.int32),
            pltpu.VMEM((NCHB, CH), jnp.float32),
            pltpu.VMEM((CH, F), jnp.float32),
            pltpu.VMEM((CH, F), jnp.float32),
            pltpu.VMEM((CH, F), jnp.float32),
            pltpu.VMEM((CH, F), jnp.float32),
            pltpu.VMEM((AWB, F), jnp.float32),
        ] + [pltpu.SemaphoreType.DMA] * 6,
    )(_make_sc_kernel(b))


def _sc_gather(b, nidx2_b, d2_b, amsij, qdn):
    return _sc_gather_built(b)(nidx2_b, d2_b, amsij, qdn)


TA = 32
RE = TA * NB


def _stage2_body(amsij_ref, qdn_ref, emlp_ref, a_ref, edyn_ref, qsum_ref,
                 aj_ref, rbf_ref, d_ref, bdyn_ref, blat_ref,
                 wr, br, wb1, bb1, wb2, bb2, wb2r, wm1, bm1, wm2, bm2,
                 aout_o, edn_o, bdn_o, bln_o):
    lane = lax.broadcasted_iota(jnp.int32, (RE, NB), 1)
    row = lax.broadcasted_iota(jnp.int32, (RE, NB), 0)
    sel = (lane == row % NB).astype(jnp.float32)

    def expand_col(x_an):
        z = jnp.broadcast_to(x_an[:, None, :], (TA, NB, NB)).reshape(RE, NB)
        return jnp.sum(z * sel, axis=1, keepdims=True)

    dv = expand_col(d_ref[...])
    x = dv * (1.0 / CUTOFF)
    x2 = x * x
    x4 = x2 * x2
    x8 = x4 * x4
    x9 = x8 * x
    x10 = x9 * x
    x11 = x10 * x
    c1 = (P + 1.0) * (P + 2.0) / 2.0
    c2 = P * (P + 2.0)
    c3 = P * (P + 1.0) / 2.0
    cut = (1.0 - c1 * x9 + c2 * x10 - c3 * x11)
    cut = cut * (dv < CUTOFF).astype(jnp.float32)

    rbfm = (jnp.dot(rbf_ref[...], wr[...]) + br[...]) * cut

    am = amsij_ref[...]
    ai = jnp.broadcast_to(am[:, None, :], (TA, NB, F)).reshape(RE, F)
    msij = ai * aj_ref[...] * rbfm

    h = _silu(jnp.dot(msij, wb1[...]) + bb1[...])
    bij = jnp.dot(h, wb2[...]) + bb2[...]
    h2 = _silu(jnp.dot(msij, wm1[...]) + bm1[...])
    m = jnp.dot(h2, wm2[...]) + bm2[...]

    bdn = bdyn_ref[...] + bij * m
    bdn_o[...] = bdn
    bij_an = (jnp.sum(h.reshape(TA, NB, F) * wb2r[...].reshape(1, 1, F),
                      axis=2) + bb2[...])
    bln_o[...] = blat_ref[...] + bij_an

    dinv = jnp.where(dv > 0.0, 1.0 / dv, 0.0)
    sb = jnp.sum((dinv * bdn).reshape(TA, NB, F), axis=1)
    de = emlp_ref[...] * (qdn_ref[...] * qsum_ref[...] - sb)
    aout_o[...] = a_ref[...] + de
    edn_o[...] = edyn_ref[...] + de


NSTEP = A // TA


@functools.cache
def _stage2_built(b):
    atom = pl.BlockSpec((TA, F), lambda i: (b * NSTEP + i, 0))
    atomnb = pl.BlockSpec((TA, NB), lambda i: (b * NSTEP + i, 0))
    edge = pl.BlockSpec((RE, F), lambda i: (b * NSTEP + i, 0))
    batom = pl.BlockSpec((TA, F), lambda i: (i, 0))
    bedge = pl.BlockSpec((RE, F), lambda i: (i, 0))
    bedger = pl.BlockSpec((RE, R), lambda i: (i, 0))
    wspec = lambda s: pl.BlockSpec(s, lambda i: (0, 0))
    return pl.pallas_call(
        _stage2_body,
        grid=(NSTEP,),
        in_specs=[atom, atom, atom, atom, atom, batom,
                  bedge, bedger, atomnb, edge, atomnb] +
                 [wspec(s) for s in
                  ((R, F), (1, F), (F, F), (1, F), (F, 1), (1, 1), (1, F),
                   (F, F), (1, F), (F, F), (1, F))],
        out_specs=[atom, atom, edge, atomnb],
        out_shape=[
            jax.ShapeDtypeStruct((M, F), jnp.float32),
            jax.ShapeDtypeStruct((M, F), jnp.float32),
            jax.ShapeDtypeStruct((E, F), jnp.float32),
            jax.ShapeDtypeStruct((M, NB), jnp.float32),
        ],
        input_output_aliases={3: 0, 4: 1, 9: 2, 10: 3},
    )


def _stage2(b, amsij, qdn, emlp, ach, ech, qsum_b, aj_b, rbff_b, df, bch, lch,
            prbf, pb, pbm):
    wr, brb = prbf
    args_w = [wr, brb.reshape(1, -1)]
    for (w1, b1, w2, b2) in (pb, pbm):
        args_w += [w1, b1.reshape(1, -1), w2, b2.reshape(1, -1)]
    args_w.insert(6, pb[2].reshape(-1)[None, :])
    return _stage2_built(b)(amsij, qdn, emlp, ach, ech, qsum_b,
                            aj_b, rbff_b, df, bch, lch, *args_w)


def kernel(a, q_dynamics, b_dynamics, e_dynamics, q_latent, b_latent,
           rbf, D, N, NM, params):
    af = a.reshape(M, F)
    qdf = q_dynamics.reshape(M, F)
    qlf = q_latent.reshape(M, 1)
    edf = e_dynamics.reshape(M, F)
    rbff = rbf.reshape(E, R)
    df = D.reshape(M, NB)
    bdf = b_dynamics.reshape(E, F)
    blf = b_latent.reshape(M, NB)
    nidx2 = N.reshape(E // CH, CH).astype(jnp.int32)
    d2 = D.reshape(E // CH, CH)
    nrow = EB // CH

    amsij, qdn, qln, emlp = _stage1(af, qdf, qlf,
                                    params['a'], params['q'],
                                    params['qm'], params['e'])

    aout, edn, bdn, bln = af, edf, bdf, blf
    for b in range(B):
        aj_b, qsum_b = _sc_gather(b, nidx2[b * nrow:(b + 1) * nrow],
                                  d2[b * nrow:(b + 1) * nrow], amsij, qdn)
        rbff_b = rbff[b * EB:(b + 1) * EB]
        aout, edn, bdn, bln = _stage2(b, amsij, qdn, emlp, aout, edn,
                                      qsum_b, aj_b, rbff_b, df, bdn, bln,
                                      params['rbf'], params['b'],
                                      params['bm'])

    return (aout.reshape(B, A, F),
            qdn.reshape(B, A, F),
            bdn.reshape(B, A, NB, F),
            edn.reshape(B, A, F),
            qln.reshape(B, A, 1),
            bln.reshape(B, A, NB))


# speedup vs baseline: 1.2192x; 1.2192x over previous
import functools

import jax
import jax.numpy as jnp
from jax import lax
from jax.experimental import pallas as pl
from jax.experimental.pallas import tpu as pltpu
from jax.experimental.pallas import tpu_sc as plsc

B, A, NB, F, R = 4, 512, 32, 128, 20
E = B * A * NB
M = B * A
CUTOFF = 5.0
P = 9

NC, NS = 2, 16
NW = NC * NS
EW = E // NW
CH = 128
NCH = EW // CH
AW = M // NW


def _sigmoid(x):
    return 1.0 / (1.0 + jnp.exp(-x))


def _silu(x):
    return x * _sigmoid(x)


TB1 = 256


def _stage1_body(a_ref, qd_ref, ql_ref,
                 wa1, ba1, wa2, ba2,
                 wq1, bq1, wq2, bq2,
                 wm1, bm1, wm2, bm2,
                 we1, be1, we2, be2,
                 amsij_o, qdn_o, qln_o, emlp_o):
    x = a_ref[...]

    def mlp(w1, b1, w2, b2):
        h = _silu(jnp.dot(x, w1[...]) + b1[...])
        return jnp.dot(h, w2[...]) + b2[...]

    amsij_o[...] = mlp(wa1, ba1, wa2, ba2)
    q = mlp(wq1, bq1, wq2, bq2)
    qm = mlp(wm1, bm1, wm2, bm2)
    qdn_o[...] = qd_ref[...] + q * qm
    qln_o[...] = ql_ref[...] + q
    emlp_o[...] = mlp(we1, be1, we2, be2)


def _stage1(af, qdf, qlf, pa, pq, pqm, pe):
    n = M // TB1
    row = pl.BlockSpec((TB1, F), lambda i: (i, 0))
    col = pl.BlockSpec((TB1, 1), lambda i: (i, 0))
    wspec = lambda s: pl.BlockSpec(s, lambda i: (0, 0))
    specs_w = []
    args_w = []
    for (w1, b1, w2, b2) in (pa, pq, pqm, pe):
        args_w += [w1, b1.reshape(1, -1), w2, b2.reshape(1, -1)]
        specs_w += [wspec(w1.shape), wspec((1, b1.shape[0])),
                    wspec(w2.shape), wspec((1, b2.shape[0]))]
    return pl.pallas_call(
        _stage1_body,
        grid=(n,),
        in_specs=[row, row, col] + specs_w,
        out_specs=[row, row, col, row],
        out_shape=[
            jax.ShapeDtypeStruct((M, F), jnp.float32),
            jax.ShapeDtypeStruct((M, F), jnp.float32),
            jax.ShapeDtypeStruct((M, 1), jnp.float32),
            jax.ShapeDtypeStruct((M, F), jnp.float32),
        ],
    )(af, qdf, qlf, *args_w)


def _lane_bcast(v16, e):
    idx = jnp.full((16, 1), e, jnp.int32)
    return lax.gather(
        v16, idx,
        lax.GatherDimensionNumbers(offset_dims=(), collapsed_slice_dims=(0,),
                                   start_index_map=(0,)),
        (1,), mode=lax.GatherScatterMode.PROMISE_IN_BOUNDS)


EB = A * NB
EWB = EB // NW
NCHB = EWB // CH
AWB = A // NW


def _make_sc_kernel(b):
    roff = b * A

    def kern(nidx2, d2, amsij, qdyn, aj_o, qsum_o,
             idx_v, w_v, ab0, ab1, qb0, qb1, qs_v,
             sga0, sga1, sgq0, sgq1, swa0, swa1):
        wid = lax.axis_index("c") * NS + lax.axis_index("s")
        ebase = wid * EWB
        abase = wid * AWB

        rbase = b * (EB // CH) + wid * NCHB
        pltpu.sync_copy(nidx2.at[pl.ds(rbase, NCHB)], idx_v)
        pltpu.sync_copy(d2.at[pl.ds(rbase, NCHB)], w_v)
        for r in range(NCHB):
            for i in range(CH // 16):
                sl = pl.ds(i * 16, 16)
                idx_v[r, sl] = idx_v[r, sl] + roff
                dd = w_v[r, sl]
                w_v[r, sl] = jnp.where(dd > 0.0, 1.0 / dd, 0.0)

        def start(c, ab, qb, sga, sgq):
            row = idx_v.at[c]
            pltpu.async_copy(amsij.at[row], ab, sga)
            pltpu.async_copy(qdyn.at[row], qb, sgq)

        start(0, ab0, qb0, sga0, sgq0)
        start(1, ab1, qb1, sga1, sgq1)

        def section(c, ab, qb, sga, sgq, swa):
            pltpu.make_async_copy(amsij.at[idx_v.at[0]], ab, sga).wait()
            pltpu.make_async_copy(qdyn.at[idx_v.at[0]], qb, sgq).wait()
            pltpu.async_copy(ab, aj_o.at[pl.ds(ebase + c * CH, CH)], swa)
            for k in range(CH // NB):
                acc = [jnp.zeros((16,), jnp.float32) for _ in range(F // 16)]
                for g in range(NB // 16):
                    w16 = w_v[c, pl.ds((k * 2 + g) * 16, 16)]
                    for e in range(16):
                        we = _lane_bcast(w16, e)
                        r = k * NB + g * 16 + e
                        for f in range(F // 16):
                            acc[f] = acc[f] + qb[r, pl.ds(f * 16, 16)] * we
                for f in range(F // 16):
                    qs_v[c * (CH // NB) + k, pl.ds(f * 16, 16)] = acc[f]
            @pl.when(c + 2 < NCHB)
            def _():
                pltpu.make_async_copy(ab, aj_o.at[pl.ds(ebase + c * CH, CH)],
                                      swa).wait()
                start(c + 2, ab, qb, sga, sgq)

        def body(co, carry):
            section(2 * co, ab0, qb0, sga0, sgq0, swa0)
            section(2 * co + 1, ab1, qb1, sga1, sgq1, swa1)
            return carry

        lax.fori_loop(0, NCHB // 2, body, 0)
        pltpu.make_async_copy(ab0, aj_o.at[pl.ds(ebase + (NCHB - 2) * CH, CH)],
                              swa0).wait()
        pltpu.make_async_copy(ab1, aj_o.at[pl.ds(ebase + (NCHB - 1) * CH, CH)],
                              swa1).wait()
        pltpu.sync_copy(qs_v, qsum_o.at[pl.ds(abase, AWB)])

    return kern


@functools.cache
def _sc_gather_built(b):
    return functools.partial(
        pl.kernel,
        mesh=plsc.VectorSubcoreMesh(core_axis_name="c", subcore_axis_name="s"),
        out_type=[
            jax.ShapeDtypeStruct((EB, F), jnp.float32),
            jax.ShapeDtypeStruct((A, F), jnp.float32),
        ],
        scratch_types=[
            pltpu.VMEM((NCHB, CH), jnp.int32),
            pltpu.VMEM((NCHB, CH), jnp.float32),
            pltpu.VMEM((CH, F), jnp.float32),
            pltpu.VMEM((CH, F), jnp.float32),
            pltpu.VMEM((CH, F), jnp.float32),
            pltpu.VMEM((CH, F), jnp.float32),
            pltpu.VMEM((AWB, F), jnp.float32),
        ] + [pltpu.SemaphoreType.DMA] * 6,
    )(_make_sc_kernel(b))


def _sc_gather(b, nidx2_b, d2_b, amsij, qdn):
    return _sc_gather_built(b)(nidx2_b, d2_b, amsij, qdn)


TA = 32
RE = TA * NB


def _stage2_body(amsij_ref, qdn_ref, emlp_ref, a_ref, edyn_ref, qsum_ref,
                 aj_ref, rbf_ref, d_ref, bdyn_ref, blat_ref,
                 wr, br, wb1, bb1, wb2, bb2, wb2r, wm1, bm1, wm2, bm2,
                 aout_o, edn_o, bdn_o, bln_o):
    lane = lax.broadcasted_iota(jnp.int32, (RE, NB), 1)
    row = lax.broadcasted_iota(jnp.int32, (RE, NB), 0)
    sel = (lane == row % NB).astype(jnp.float32)

    def expand_col(x_an):
        z = jnp.broadcast_to(x_an[:, None, :], (TA, NB, NB)).reshape(RE, NB)
        return jnp.sum(z * sel, axis=1, keepdims=True)

    dv = expand_col(d_ref[...])
    x = dv * (1.0 / CUTOFF)
    x2 = x * x
    x4 = x2 * x2
    x8 = x4 * x4
    x9 = x8 * x
    x10 = x9 * x
    x11 = x10 * x
    c1 = (P + 1.0) * (P + 2.0) / 2.0
    c2 = P * (P + 2.0)
    c3 = P * (P + 1.0) / 2.0
    cut = (1.0 - c1 * x9 + c2 * x10 - c3 * x11)
    cut = cut * (dv < CUTOFF).astype(jnp.float32)

    rbfm = (jnp.dot(rbf_ref[...], wr[...]) + br[...]) * cut

    am = amsij_ref[...]
    ai = jnp.broadcast_to(am[:, None, :], (TA, NB, F)).reshape(RE, F)
    msij = ai * aj_ref[...] * rbfm

    h = _silu(jnp.dot(msij, wb1[...]) + bb1[...])
    bij = jnp.dot(h, wb2[...]) + bb2[...]
    h2 = _silu(jnp.dot(msij, wm1[...]) + bm1[...])
    m = jnp.dot(h2, wm2[...]) + bm2[...]

    bdn = bdyn_ref[...] + bij * m
    bdn_o[...] = bdn
    bij_an = (jnp.sum(h.reshape(TA, NB, F) * wb2r[...].reshape(1, 1, F),
                      axis=2) + bb2[...])
    bln_o[...] = blat_ref[...] + bij_an

    dinv = jnp.where(dv > 0.0, 1.0 / dv, 0.0)
    sb = jnp.sum((dinv * bdn).reshape(TA, NB, F), axis=1)
    de = emlp_ref[...] * (qdn_ref[...] * qsum_ref[...] - sb)
    aout_o[...] = a_ref[...] + de
    edn_o[...] = edyn_ref[...] + de


NSTEP = A // TA


def _stage2_chain_body(*refs):
    _stage2_body(*refs[:22], *refs[-4:])


@functools.cache
def _stage2_built(b):
    atom = pl.BlockSpec((TA, F), lambda i: (b * NSTEP + i, 0))
    atomnb = pl.BlockSpec((TA, NB), lambda i: (b * NSTEP + i, 0))
    edge = pl.BlockSpec((RE, F), lambda i: (b * NSTEP + i, 0))
    batom = pl.BlockSpec((TA, F), lambda i: (i, 0))
    bedge = pl.BlockSpec((RE, F), lambda i: (i, 0))
    bedger = pl.BlockSpec((RE, R), lambda i: (i, 0))
    wspec = lambda s: pl.BlockSpec(s, lambda i: (0, 0))
    touch = lambda s: pl.BlockSpec(s, lambda i: (0, 0))
    in_specs = [atom, atom, atom, atom, atom, batom,
                bedge, bedger, atomnb, edge, atomnb] + \
               [wspec(s) for s in
                ((R, F), (1, F), (F, F), (1, F), (F, 1), (1, 1), (1, F),
                 (F, F), (1, F), (F, F), (1, F))]
    body = _stage2_body
    aliases = {}
    if b > 0:
        in_specs = in_specs + [touch((8, F)), touch((8, F)),
                               touch((8, F)), touch((8, NB))]
        body = _stage2_chain_body
        aliases = {22: 0, 23: 1, 24: 2, 25: 3}
    return pl.pallas_call(
        body,
        grid=(NSTEP,),
        in_specs=in_specs,
        out_specs=[atom, atom, edge, atomnb],
        out_shape=[
            jax.ShapeDtypeStruct((M, F), jnp.float32),
            jax.ShapeDtypeStruct((M, F), jnp.float32),
            jax.ShapeDtypeStruct((E, F), jnp.float32),
            jax.ShapeDtypeStruct((M, NB), jnp.float32),
        ],
        input_output_aliases=aliases,
    )


def _stage2(b, amsij, qdn, emlp, af, edf, qsum_b, aj_b, rbff_b, df, bdf, blf,
            chain, prbf, pb, pbm):
    wr, brb = prbf
    args_w = [wr, brb.reshape(1, -1)]
    for (w1, b1, w2, b2) in (pb, pbm):
        args_w += [w1, b1.reshape(1, -1), w2, b2.reshape(1, -1)]
    args_w.insert(6, pb[2].reshape(-1)[None, :])
    args = [amsij, qdn, emlp, af, edf, qsum_b, aj_b, rbff_b, df, bdf, blf,
            *args_w]
    if b > 0:
        args += list(chain)
    return _stage2_built(b)(*args)


def kernel(a, q_dynamics, b_dynamics, e_dynamics, q_latent, b_latent,
           rbf, D, N, NM, params):
    af = a.reshape(M, F)
    qdf = q_dynamics.reshape(M, F)
    qlf = q_latent.reshape(M, 1)
    edf = e_dynamics.reshape(M, F)
    df = D.reshape(M, NB)
    bdf = b_dynamics.reshape(E, F)
    blf = b_latent.reshape(M, NB)
    nidx2 = N.reshape(E // CH, CH).astype(jnp.int32)
    d2 = D.reshape(E // CH, CH)

    amsij, qdn, qln, emlp = _stage1(af, qdf, qlf,
                                    params['a'], params['q'],
                                    params['qm'], params['e'])

    chain = None
    for b in range(B):
        aj_b, qsum_b = _sc_gather(b, nidx2, d2, amsij, qdn)
        rbff_b = rbf[b].reshape(EB, R)
        chain = _stage2(b, amsij, qdn, emlp, af, edf,
                        qsum_b, aj_b, rbff_b, df, bdf, blf, chain,
                        params['rbf'], params['b'], params['bm'])
    aout, edn, bdn, bln = chain

    return (aout.reshape(B, A, F),
            qdn.reshape(B, A, F),
            bdn.reshape(B, A, NB, F),
            edn.reshape(B, A, F),
            qln.reshape(B, A, 1),
            bln.reshape(B, A, NB))


# speedup vs baseline: 1.3162x; 1.0795x over previous
import functools

import jax
import jax.numpy as jnp
from jax import lax
from jax.experimental import pallas as pl
from jax.experimental.pallas import tpu as pltpu
from jax.experimental.pallas import tpu_sc as plsc

B, A, NB, F, R = 4, 512, 32, 128, 20
E = B * A * NB
M = B * A
CUTOFF = 5.0
P = 9

NC, NS = 2, 16
NW = NC * NS
EW = E // NW
CH = 128
NCH = EW // CH
AW = M // NW


def _sigmoid(x):
    return 1.0 / (1.0 + jnp.exp(-x))


def _silu(x):
    return x * _sigmoid(x)


TB1 = 256


def _stage1_body(a_ref, qd_ref, ql_ref,
                 wa1, ba1, wa2, ba2,
                 wq1, bq1, wq2, bq2,
                 wm1, bm1, wm2, bm2,
                 we1, be1, we2, be2,
                 amsij_o, amsb_o, qdn_o, qln_o, emlp_o):
    x = a_ref[...]

    def mlp(w1, b1, w2, b2):
        h = _silu(jnp.dot(x, w1[...]) + b1[...])
        return jnp.dot(h, w2[...]) + b2[...]

    am = mlp(wa1, ba1, wa2, ba2)
    amsij_o[...] = am
    amsb_o[...] = am.astype(jnp.bfloat16)
    q = mlp(wq1, bq1, wq2, bq2)
    qm = mlp(wm1, bm1, wm2, bm2)
    qdn_o[...] = qd_ref[...] + q * qm
    qln_o[...] = ql_ref[...] + q
    emlp_o[...] = mlp(we1, be1, we2, be2)


def _stage1(af, qdf, qlf, pa, pq, pqm, pe):
    n = M // TB1
    row = pl.BlockSpec((TB1, F), lambda i: (i, 0))
    col = pl.BlockSpec((TB1, 1), lambda i: (i, 0))
    wspec = lambda s: pl.BlockSpec(s, lambda i: (0, 0))
    specs_w = []
    args_w = []
    for (w1, b1, w2, b2) in (pa, pq, pqm, pe):
        args_w += [w1, b1.reshape(1, -1), w2, b2.reshape(1, -1)]
        specs_w += [wspec(w1.shape), wspec((1, b1.shape[0])),
                    wspec(w2.shape), wspec((1, b2.shape[0]))]
    return pl.pallas_call(
        _stage1_body,
        grid=(n,),
        in_specs=[row, row, col] + specs_w,
        out_specs=[row, row, row, col, row],
        out_shape=[
            jax.ShapeDtypeStruct((M, F), jnp.float32),
            jax.ShapeDtypeStruct((M, F), jnp.bfloat16),
            jax.ShapeDtypeStruct((M, F), jnp.float32),
            jax.ShapeDtypeStruct((M, 1), jnp.float32),
            jax.ShapeDtypeStruct((M, F), jnp.float32),
        ],
    )(af, qdf, qlf, *args_w)


def _lane_bcast(v16, e):
    idx = jnp.full((16, 1), e, jnp.int32)
    return lax.gather(
        v16, idx,
        lax.GatherDimensionNumbers(offset_dims=(), collapsed_slice_dims=(0,),
                                   start_index_map=(0,)),
        (1,), mode=lax.GatherScatterMode.PROMISE_IN_BOUNDS)


EB = A * NB
EWB = EB // NW
NCHB = EWB // CH
AWB = A // NW


def _make_sc_kernel(b):
    roff = b * A

    def kern(nidx2, d2, qdyn, qsum_o,
             idx_v, w_v, qb0, qb1, qs_v, sgq0, sgq1):
        wid = lax.axis_index("c") * NS + lax.axis_index("s")
        abase = wid * AWB

        rbase = b * (EB // CH) + wid * NCHB
        pltpu.sync_copy(nidx2.at[pl.ds(rbase, NCHB)], idx_v)
        pltpu.sync_copy(d2.at[pl.ds(rbase, NCHB)], w_v)
        for r in range(NCHB):
            for i in range(CH // 16):
                sl = pl.ds(i * 16, 16)
                idx_v[r, sl] = idx_v[r, sl] + roff
                dd = w_v[r, sl]
                w_v[r, sl] = jnp.where(dd > 0.0, 1.0 / dd, 0.0)

        def start(c, qb, sgq):
            pltpu.async_copy(qdyn.at[idx_v.at[c]], qb, sgq)

        start(0, qb0, sgq0)
        start(1, qb1, sgq1)

        def section(c, qb, sgq):
            pltpu.make_async_copy(qdyn.at[idx_v.at[0]], qb, sgq).wait()
            for k in range(CH // NB):
                acc = [jnp.zeros((16,), jnp.float32) for _ in range(F // 16)]
                for g in range(NB // 16):
                    w16 = w_v[c, pl.ds((k * 2 + g) * 16, 16)]
                    for e in range(16):
                        we = _lane_bcast(w16, e)
                        r = k * NB + g * 16 + e
                        for f in range(F // 16):
                            acc[f] = acc[f] + qb[r, pl.ds(f * 16, 16)] * we
                for f in range(F // 16):
                    qs_v[c * (CH // NB) + k, pl.ds(f * 16, 16)] = acc[f]
            @pl.when(c + 2 < NCHB)
            def _():
                start(c + 2, qb, sgq)

        def body(co, carry):
            section(2 * co, qb0, sgq0)
            section(2 * co + 1, qb1, sgq1)
            return carry

        lax.fori_loop(0, NCHB // 2, body, 0)
        pltpu.sync_copy(qs_v, qsum_o.at[pl.ds(abase, AWB)])

    return kern


@functools.cache
def _sc_gather_built(b):
    return functools.partial(
        pl.kernel,
        mesh=plsc.VectorSubcoreMesh(core_axis_name="c", subcore_axis_name="s"),
        out_type=jax.ShapeDtypeStruct((A, F), jnp.float32),
        scratch_types=[
            pltpu.VMEM((NCHB, CH), jnp.int32),
            pltpu.VMEM((NCHB, CH), jnp.float32),
            pltpu.VMEM((CH, F), jnp.float32),
            pltpu.VMEM((CH, F), jnp.float32),
            pltpu.VMEM((AWB, F), jnp.float32),
        ] + [pltpu.SemaphoreType.DMA] * 2,
    )(_make_sc_kernel(b))


def _sc_gather(b, nidx2, d2, qdn):
    return _sc_gather_built(b)(nidx2, d2, qdn)


TA = 32
RE = TA * NB


def _stage2_body(amsij_ref, qdn_ref, emlp_ref, a_ref, edyn_ref, qsum_ref,
                 amsb_ref, nidx_ref, rbf_ref, d_ref, bdyn_ref, blat_ref,
                 wr, br, wb1, bb1, wb2, bb2, wb2r, wm1, bm1, wm2, bm2,
                 aout_o, edn_o, bdn_o, bln_o):
    lane = lax.broadcasted_iota(jnp.int32, (RE, NB), 1)
    row = lax.broadcasted_iota(jnp.int32, (RE, NB), 0)
    sel = (lane == row % NB).astype(jnp.float32)

    def expand_col(x_an):
        z = jnp.broadcast_to(x_an[:, None, :], (TA, NB, NB)).reshape(RE, NB)
        return jnp.sum(z * sel, axis=1, keepdims=True)

    dv = expand_col(d_ref[...])
    x = dv * (1.0 / CUTOFF)
    x2 = x * x
    x4 = x2 * x2
    x8 = x4 * x4
    x9 = x8 * x
    x10 = x9 * x
    x11 = x10 * x
    c1 = (P + 1.0) * (P + 2.0) / 2.0
    c2 = P * (P + 2.0)
    c3 = P * (P + 1.0) / 2.0
    cut = (1.0 - c1 * x9 + c2 * x10 - c3 * x11)
    cut = cut * (dv < CUTOFF).astype(jnp.float32)

    rbfm = (jnp.dot(rbf_ref[...], wr[...]) + br[...]) * cut

    am = amsij_ref[...]
    ai = jnp.broadcast_to(am[:, None, :], (TA, NB, F)).reshape(RE, F)
    ncol = expand_col(nidx_ref[...].astype(jnp.float32))
    vals = lax.broadcasted_iota(jnp.int32, (RE, A), 1)
    oh = (vals == ncol.astype(jnp.int32)).astype(jnp.bfloat16)
    ajv = jnp.dot(oh, amsb_ref[...],
                  preferred_element_type=jnp.float32)
    msij = ai * ajv * rbfm

    h = _silu(jnp.dot(msij, wb1[...]) + bb1[...])
    bij = jnp.dot(h, wb2[...]) + bb2[...]
    h2 = _silu(jnp.dot(msij, wm1[...]) + bm1[...])
    m = jnp.dot(h2, wm2[...]) + bm2[...]

    bdn = bdyn_ref[...] + bij * m
    bdn_o[...] = bdn
    bij_an = (jnp.sum(h.reshape(TA, NB, F) * wb2r[...].reshape(1, 1, F),
                      axis=2) + bb2[...])
    bln_o[...] = blat_ref[...] + bij_an

    dinv = jnp.where(dv > 0.0, 1.0 / dv, 0.0)
    sb = jnp.sum((dinv * bdn).reshape(TA, NB, F), axis=1)
    de = emlp_ref[...] * (qdn_ref[...] * qsum_ref[...] - sb)
    aout_o[...] = a_ref[...] + de
    edn_o[...] = edyn_ref[...] + de


NSTEP = A // TA


def _stage2_chain_body(*refs):
    _stage2_body(*refs[:23], *refs[-4:])


@functools.cache
def _stage2_built(b):
    atom = pl.BlockSpec((TA, F), lambda i: (b * NSTEP + i, 0))
    atomnb = pl.BlockSpec((TA, NB), lambda i: (b * NSTEP + i, 0))
    edge = pl.BlockSpec((RE, F), lambda i: (b * NSTEP + i, 0))
    batom = pl.BlockSpec((TA, F), lambda i: (i, 0))
    btable = pl.BlockSpec((A, F), lambda i: (b, 0))
    bedger = pl.BlockSpec((RE, R), lambda i: (i, 0))
    wspec = lambda s: pl.BlockSpec(s, lambda i: (0, 0))
    touch = lambda s: pl.BlockSpec(s, lambda i: (0, 0))
    in_specs = [atom, atom, atom, atom, atom, batom,
                btable, atomnb, bedger, atomnb, edge, atomnb] + \
               [wspec(s) for s in
                ((R, F), (1, F), (F, F), (1, F), (F, 1), (1, 1), (1, F),
                 (F, F), (1, F), (F, F), (1, F))]
    body = _stage2_body
    aliases = {}
    if b > 0:
        in_specs = in_specs + [touch((8, F)), touch((8, F)),
                               touch((8, F)), touch((8, NB))]
        body = _stage2_chain_body
        aliases = {23: 0, 24: 1, 25: 2, 26: 3}
    return pl.pallas_call(
        body,
        grid=(NSTEP,),
        in_specs=in_specs,
        out_specs=[atom, atom, edge, atomnb],
        out_shape=[
            jax.ShapeDtypeStruct((M, F), jnp.float32),
            jax.ShapeDtypeStruct((M, F), jnp.float32),
            jax.ShapeDtypeStruct((E, F), jnp.float32),
            jax.ShapeDtypeStruct((M, NB), jnp.float32),
        ],
        input_output_aliases=aliases,
    )


def _stage2(b, amsij, qdn, emlp, af, edf, qsum_b, amsb, n32, rbff_b, df,
            bdf, blf, chain, prbf, pb, pbm):
    wr, brb = prbf
    args_w = [wr, brb.reshape(1, -1)]
    for (w1, b1, w2, b2) in (pb, pbm):
        args_w += [w1, b1.reshape(1, -1), w2, b2.reshape(1, -1)]
    args_w.insert(6, pb[2].reshape(-1)[None, :])
    args = [amsij, qdn, emlp, af, edf, qsum_b, amsb, n32, rbff_b, df,
            bdf, blf, *args_w]
    if b > 0:
        args += list(chain)
    return _stage2_built(b)(*args)


def kernel(a, q_dynamics, b_dynamics, e_dynamics, q_latent, b_latent,
           rbf, D, N, NM, params):
    af = a.reshape(M, F)
    qdf = q_dynamics.reshape(M, F)
    qlf = q_latent.reshape(M, 1)
    edf = e_dynamics.reshape(M, F)
    df = D.reshape(M, NB)
    bdf = b_dynamics.reshape(E, F)
    blf = b_latent.reshape(M, NB)
    nidx2 = N.reshape(E // CH, CH).astype(jnp.int32)
    d2 = D.reshape(E // CH, CH)
    n32 = N.reshape(M, NB).astype(jnp.int32)

    amsij, amsb, qdn, qln, emlp = _stage1(af, qdf, qlf,
                                          params['a'], params['q'],
                                          params['qm'], params['e'])

    chain = None
    for b in range(B):
        qsum_b = _sc_gather(b, nidx2, d2, qdn)
        rbff_b = rbf[b].reshape(EB, R)
        chain = _stage2(b, amsij, qdn, emlp, af, edf,
                        qsum_b, amsb, n32, rbff_b, df, bdf, blf, chain,
                        params['rbf'], params['b'], params['bm'])
    aout, edn, bdn, bln = chain

    return (aout.reshape(B, A, F),
            qdn.reshape(B, A, F),
            bdn.reshape(B, A, NB, F),
            edn.reshape(B, A, F),
            qln.reshape(B, A, 1),
            bln.reshape(B, A, NB))


# speedup vs baseline: 1.3842x; 1.0516x over previous
import functools

import jax
import jax.numpy as jnp
from jax import lax
from jax.experimental import pallas as pl
from jax.experimental.pallas import tpu as pltpu
from jax.experimental.pallas import tpu_sc as plsc

B, A, NB, F, R = 4, 512, 32, 128, 20
E = B * A * NB
M = B * A
CUTOFF = 5.0
P = 9

NC, NS = 2, 16
NW = NC * NS
EW = E // NW
CH = 128
NCH = EW // CH
AW = M // NW


def _sigmoid(x):
    return 1.0 / (1.0 + jnp.exp(-x))


def _silu(x):
    return x * _sigmoid(x)


TB1 = 256


def _stage1_body(a_ref, qd_ref, ql_ref,
                 wa1, ba1, wa2, ba2,
                 wq1, bq1, wq2, bq2,
                 wm1, bm1, wm2, bm2,
                 we1, be1, we2, be2,
                 amsij_o, amsb_o, qdn_o, qln_o, emlp_o):
    x = a_ref[...]

    def mlp(w1, b1, w2, b2):
        h = _silu(jnp.dot(x, w1[...]) + b1[...])
        return jnp.dot(h, w2[...]) + b2[...]

    am = mlp(wa1, ba1, wa2, ba2)
    amsij_o[...] = am
    amsb_o[...] = am.astype(jnp.bfloat16)
    q = mlp(wq1, bq1, wq2, bq2)
    qm = mlp(wm1, bm1, wm2, bm2)
    qdn_o[...] = qd_ref[...] + q * qm
    qln_o[...] = ql_ref[...] + q
    emlp_o[...] = mlp(we1, be1, we2, be2)


def _stage1(af, qdf, qlf, pa, pq, pqm, pe):
    n = M // TB1
    row = pl.BlockSpec((TB1, F), lambda i: (i, 0))
    col = pl.BlockSpec((TB1, 1), lambda i: (i, 0))
    wspec = lambda s: pl.BlockSpec(s, lambda i: (0, 0))
    specs_w = []
    args_w = []
    for (w1, b1, w2, b2) in (pa, pq, pqm, pe):
        args_w += [w1, b1.reshape(1, -1), w2, b2.reshape(1, -1)]
        specs_w += [wspec(w1.shape), wspec((1, b1.shape[0])),
                    wspec(w2.shape), wspec((1, b2.shape[0]))]
    return pl.pallas_call(
        _stage1_body,
        grid=(n,),
        in_specs=[row, row, col] + specs_w,
        out_specs=[row, row, row, col, row],
        out_shape=[
            jax.ShapeDtypeStruct((M, F), jnp.float32),
            jax.ShapeDtypeStruct((M, F), jnp.bfloat16),
            jax.ShapeDtypeStruct((M, F), jnp.float32),
            jax.ShapeDtypeStruct((M, 1), jnp.float32),
            jax.ShapeDtypeStruct((M, F), jnp.float32),
        ],
    )(af, qdf, qlf, *args_w)


def _lane_bcast(v16, e):
    idx = jnp.full((16, 1), e, jnp.int32)
    return lax.gather(
        v16, idx,
        lax.GatherDimensionNumbers(offset_dims=(), collapsed_slice_dims=(0,),
                                   start_index_map=(0,)),
        (1,), mode=lax.GatherScatterMode.PROMISE_IN_BOUNDS)


def _sc_qsum_kernel(nidx2, d2, qdyn, qsum_o,
                    idx_v, w_v, qb0, qb1, qs_v, sgq0, sgq1):
    wid = lax.axis_index("c") * NS + lax.axis_index("s")
    abase = wid * AW
    roff = (wid // (NW // B)) * A

    pltpu.sync_copy(nidx2.at[pl.ds(wid * NCH, NCH)], idx_v)
    pltpu.sync_copy(d2.at[pl.ds(wid * NCH, NCH)], w_v)
    for r in range(NCH):
        for i in range(CH // 16):
            sl = pl.ds(i * 16, 16)
            idx_v[r, sl] = idx_v[r, sl] + roff
            dd = w_v[r, sl]
            w_v[r, sl] = jnp.where(dd > 0.0, 1.0 / dd, 0.0)

    def start(c, qb, sgq):
        pltpu.async_copy(qdyn.at[idx_v.at[c]], qb, sgq)

    start(0, qb0, sgq0)
    start(1, qb1, sgq1)

    def section(c, qb, sgq):
        pltpu.make_async_copy(qdyn.at[idx_v.at[0]], qb, sgq).wait()
        for k in range(CH // NB):
            acc = [jnp.zeros((16,), jnp.float32) for _ in range(F // 16)]
            for g in range(NB // 16):
                w16 = w_v[c, pl.ds((k * 2 + g) * 16, 16)]
                for e in range(16):
                    we = _lane_bcast(w16, e)
                    r = k * NB + g * 16 + e
                    for f in range(F // 16):
                        acc[f] = acc[f] + qb[r, pl.ds(f * 16, 16)] * we
            for f in range(F // 16):
                qs_v[c * (CH // NB) + k, pl.ds(f * 16, 16)] = acc[f]
        @pl.when(c + 2 < NCH)
        def _():
            start(c + 2, qb, sgq)

    def body(co, carry):
        section(2 * co, qb0, sgq0)
        section(2 * co + 1, qb1, sgq1)
        return carry

    lax.fori_loop(0, NCH // 2, body, 0)
    pltpu.sync_copy(qs_v, qsum_o.at[pl.ds(abase, AW)])


@functools.cache
def _sc_gather_built():
    return functools.partial(
        pl.kernel,
        mesh=plsc.VectorSubcoreMesh(core_axis_name="c", subcore_axis_name="s"),
        out_type=jax.ShapeDtypeStruct((M, F), jnp.float32),
        scratch_types=[
            pltpu.VMEM((NCH, CH), jnp.int32),
            pltpu.VMEM((NCH, CH), jnp.float32),
            pltpu.VMEM((CH, F), jnp.float32),
            pltpu.VMEM((CH, F), jnp.float32),
            pltpu.VMEM((AW, F), jnp.float32),
        ] + [pltpu.SemaphoreType.DMA] * 2,
    )(_sc_qsum_kernel)


def _sc_gather(nidx2, d2, qdn):
    return _sc_gather_built()(nidx2, d2, qdn)


TA = 32
RE = TA * NB


def _stage2_body(amsij_ref, qdn_ref, emlp_ref, a_ref, edyn_ref, qsum_ref,
                 amsb_ref, nidx_ref, rbf_ref, d_ref, bdyn_ref, blat_ref,
                 wr, br, wb1, bb1, wb2, bb2, wb2r, wm1, bm1, wm2, bm2,
                 aout_o, edn_o, bdn_o, bln_o):
    lane = lax.broadcasted_iota(jnp.int32, (RE, NB), 1)
    row = lax.broadcasted_iota(jnp.int32, (RE, NB), 0)
    sel = (lane == row % NB).astype(jnp.float32)

    def expand_col(x_an):
        z = jnp.broadcast_to(x_an[:, None, :], (TA, NB, NB)).reshape(RE, NB)
        return jnp.sum(z * sel, axis=1, keepdims=True)

    dv = expand_col(d_ref[...])
    x = dv * (1.0 / CUTOFF)
    x2 = x * x
    x4 = x2 * x2
    x8 = x4 * x4
    x9 = x8 * x
    x10 = x9 * x
    x11 = x10 * x
    c1 = (P + 1.0) * (P + 2.0) / 2.0
    c2 = P * (P + 2.0)
    c3 = P * (P + 1.0) / 2.0
    cut = (1.0 - c1 * x9 + c2 * x10 - c3 * x11)
    cut = cut * (dv < CUTOFF).astype(jnp.float32)

    rbfm = (jnp.dot(rbf_ref[...], wr[...]) + br[...]) * cut

    am = amsij_ref[...]
    ai = jnp.broadcast_to(am[:, None, :], (TA, NB, F)).reshape(RE, F)
    ncol = expand_col(nidx_ref[...].astype(jnp.float32))
    vals = lax.broadcasted_iota(jnp.int32, (RE, A), 1)
    oh = (vals == ncol.astype(jnp.int32)).astype(jnp.bfloat16)
    ajv = jnp.dot(oh, amsb_ref[...],
                  preferred_element_type=jnp.float32)
    msij = ai * ajv * rbfm

    h = _silu(jnp.dot(msij, wb1[...]) + bb1[...])
    bij = jnp.dot(h, wb2[...]) + bb2[...]
    h2 = _silu(jnp.dot(msij, wm1[...]) + bm1[...])
    m = jnp.dot(h2, wm2[...]) + bm2[...]

    bdn = bdyn_ref[...] + bij * m
    bdn_o[...] = bdn
    bij_an = (jnp.sum(h.reshape(TA, NB, F) * wb2r[...].reshape(1, 1, F),
                      axis=2) + bb2[...])
    bln_o[...] = blat_ref[...] + bij_an

    dinv = jnp.where(dv > 0.0, 1.0 / dv, 0.0)
    sb = jnp.sum((dinv * bdn).reshape(TA, NB, F), axis=1)
    de = emlp_ref[...] * (qdn_ref[...] * qsum_ref[...] - sb)
    aout_o[...] = a_ref[...] + de
    edn_o[...] = edyn_ref[...] + de


NSTEP = A // TA


@functools.cache
def _stage2_built():
    atom = pl.BlockSpec((TA, F), lambda i: (i, 0))
    atomnb = pl.BlockSpec((TA, NB), lambda i: (i, 0))
    edge = pl.BlockSpec((RE, F), lambda i: (i, 0))
    table = pl.BlockSpec((A, F), lambda i: (i // NSTEP, 0))
    edger = pl.BlockSpec((RE, R), lambda i: (i, 0))
    wspec = lambda s: pl.BlockSpec(s, lambda i: (0, 0))
    return pl.pallas_call(
        _stage2_body,
        grid=(M // TA,),
        in_specs=[atom, atom, atom, atom, atom, atom,
                  table, atomnb, edger, atomnb, edge, atomnb] +
                 [wspec(s) for s in
                  ((R, F), (1, F), (F, F), (1, F), (F, 1), (1, 1), (1, F),
                   (F, F), (1, F), (F, F), (1, F))],
        out_specs=[atom, atom, edge, atomnb],
        out_shape=[
            jax.ShapeDtypeStruct((M, F), jnp.float32),
            jax.ShapeDtypeStruct((M, F), jnp.float32),
            jax.ShapeDtypeStruct((E, F), jnp.float32),
            jax.ShapeDtypeStruct((M, NB), jnp.float32),
        ],
    )


def _stage2(amsij, qdn, emlp, af, edf, qsum, amsb, n32, rbff, df,
            bdf, blf, prbf, pb, pbm):
    wr, brb = prbf
    args_w = [wr, brb.reshape(1, -1)]
    for (w1, b1, w2, b2) in (pb, pbm):
        args_w += [w1, b1.reshape(1, -1), w2, b2.reshape(1, -1)]
    args_w.insert(6, pb[2].reshape(-1)[None, :])
    return _stage2_built()(amsij, qdn, emlp, af, edf, qsum, amsb, n32,
                           rbff, df, bdf, blf, *args_w)


def kernel(a, q_dynamics, b_dynamics, e_dynamics, q_latent, b_latent,
           rbf, D, N, NM, params):
    af = a.reshape(M, F)
    qdf = q_dynamics.reshape(M, F)
    qlf = q_latent.reshape(M, 1)
    edf = e_dynamics.reshape(M, F)
    df = D.reshape(M, NB)
    bdf = b_dynamics.reshape(E, F)
    blf = b_latent.reshape(M, NB)
    nidx2 = N.reshape(E // CH, CH).astype(jnp.int32)
    d2 = D.reshape(E // CH, CH)
    n32 = N.reshape(M, NB).astype(jnp.int32)

    amsij, amsb, qdn, qln, emlp = _stage1(af, qdf, qlf,
                                          params['a'], params['q'],
                                          params['qm'], params['e'])

    qsum = _sc_gather(nidx2, d2, qdn)
    rbff = rbf.reshape(E, R)
    aout, edn, bdn, bln = _stage2(amsij, qdn, emlp, af, edf,
                                  qsum, amsb, n32, rbff, df, bdf, blf,
                                  params['rbf'], params['b'], params['bm'])

    return (aout.reshape(B, A, F),
            qdn.reshape(B, A, F),
            bdn.reshape(B, A, NB, F),
            edn.reshape(B, A, F),
            qln.reshape(B, A, 1),
            bln.reshape(B, A, NB))


# speedup vs baseline: 1.4290x; 1.0324x over previous
import functools

import jax
import jax.numpy as jnp
from jax import lax
from jax.experimental import pallas as pl
from jax.experimental.pallas import tpu as pltpu
from jax.experimental.pallas import tpu_sc as plsc

B, A, NB, F, R = 4, 512, 32, 128, 20
E = B * A * NB
M = B * A
CUTOFF = 5.0
P = 9

NC, NS = 2, 16
NW = NC * NS
EW = E // NW
CH = 128
NCH = EW // CH
AW = M // NW


def _sigmoid(x):
    return 1.0 / (1.0 + jnp.exp(-x))


def _silu(x):
    return x * _sigmoid(x)


TB1 = 256


def _stage1_body(a_ref, qd_ref, ql_ref,
                 wa1, ba1, wa2, ba2,
                 wq1, bq1, wq2, bq2,
                 wm1, bm1, wm2, bm2,
                 we1, be1, we2, be2,
                 amsij_o, amsb_o, qdn_o, qln_o, emlp_o):
    x = a_ref[...]

    def mlp(w1, b1, w2, b2):
        h = _silu(jnp.dot(x, w1[...]) + b1[...])
        return jnp.dot(h, w2[...]) + b2[...]

    am = mlp(wa1, ba1, wa2, ba2)
    amsij_o[...] = am
    amsb_o[...] = am.astype(jnp.bfloat16)
    q = mlp(wq1, bq1, wq2, bq2)
    qm = mlp(wm1, bm1, wm2, bm2)
    qdn_o[...] = qd_ref[...] + q * qm
    qln_o[...] = ql_ref[...] + q
    emlp_o[...] = mlp(we1, be1, we2, be2)


def _stage1(af, qdf, qlf, pa, pq, pqm, pe):
    n = M // TB1
    row = pl.BlockSpec((TB1, F), lambda i: (i, 0))
    col = pl.BlockSpec((TB1, 1), lambda i: (i, 0))
    wspec = lambda s: pl.BlockSpec(s, lambda i: (0, 0))
    specs_w = []
    args_w = []
    for (w1, b1, w2, b2) in (pa, pq, pqm, pe):
        args_w += [w1, b1.reshape(1, -1), w2, b2.reshape(1, -1)]
        specs_w += [wspec(w1.shape), wspec((1, b1.shape[0])),
                    wspec(w2.shape), wspec((1, b2.shape[0]))]
    return pl.pallas_call(
        _stage1_body,
        grid=(n,),
        in_specs=[row, row, col] + specs_w,
        out_specs=[row, row, row, col, row],
        out_shape=[
            jax.ShapeDtypeStruct((M, F), jnp.float32),
            jax.ShapeDtypeStruct((M, F), jnp.bfloat16),
            jax.ShapeDtypeStruct((M, F), jnp.float32),
            jax.ShapeDtypeStruct((M, 1), jnp.float32),
            jax.ShapeDtypeStruct((M, F), jnp.float32),
        ],
    )(af, qdf, qlf, *args_w)


def _lane_bcast(v16, e):
    idx = jnp.full((16, 1), e, jnp.int32)
    return lax.gather(
        v16, idx,
        lax.GatherDimensionNumbers(offset_dims=(), collapsed_slice_dims=(0,),
                                   start_index_map=(0,)),
        (1,), mode=lax.GatherScatterMode.PROMISE_IN_BOUNDS)


def _sc_qsum_kernel(nidx2, d2, qdyn, qsum_o,
                    idx_v, w_v, qb0, qb1, qs_v, sgq0, sgq1):
    wid = lax.axis_index("c") * NS + lax.axis_index("s")
    abase = wid * AW
    roff = (wid // (NW // B)) * A

    pltpu.sync_copy(nidx2.at[pl.ds(wid * NCH, NCH)], idx_v)
    pltpu.sync_copy(d2.at[pl.ds(wid * NCH, NCH)], w_v)
    for r in range(NCH):
        for i in range(CH // 16):
            sl = pl.ds(i * 16, 16)
            idx_v[r, sl] = idx_v[r, sl] + roff
            dd = w_v[r, sl]
            w_v[r, sl] = jnp.where(dd > 0.0, 1.0 / dd, 0.0)

    def start(c, qb, sgq):
        pltpu.async_copy(qdyn.at[idx_v.at[c]], qb, sgq)

    start(0, qb0, sgq0)
    start(1, qb1, sgq1)

    def section(c, qb, sgq):
        pltpu.make_async_copy(qdyn.at[idx_v.at[0]], qb, sgq).wait()
        for k in range(CH // NB):
            acc = [jnp.zeros((16,), jnp.float32) for _ in range(F // 16)]
            for g in range(NB // 16):
                w16 = w_v[c, pl.ds((k * 2 + g) * 16, 16)]
                for e in range(16):
                    we = _lane_bcast(w16, e)
                    r = k * NB + g * 16 + e
                    for f in range(F // 16):
                        acc[f] = acc[f] + qb[r, pl.ds(f * 16, 16)] * we
            for f in range(F // 16):
                qs_v[c * (CH // NB) + k, pl.ds(f * 16, 16)] = acc[f]
        @pl.when(c + 2 < NCH)
        def _():
            start(c + 2, qb, sgq)

    def body(co, carry):
        section(2 * co, qb0, sgq0)
        section(2 * co + 1, qb1, sgq1)
        return carry

    lax.fori_loop(0, NCH // 2, body, 0)
    pltpu.sync_copy(qs_v, qsum_o.at[pl.ds(abase, AW)])


@functools.cache
def _sc_gather_built():
    return functools.partial(
        pl.kernel,
        mesh=plsc.VectorSubcoreMesh(core_axis_name="c", subcore_axis_name="s"),
        out_type=jax.ShapeDtypeStruct((M, F), jnp.float32),
        scratch_types=[
            pltpu.VMEM((NCH, CH), jnp.int32),
            pltpu.VMEM((NCH, CH), jnp.float32),
            pltpu.VMEM((CH, F), jnp.float32),
            pltpu.VMEM((CH, F), jnp.float32),
            pltpu.VMEM((AW, F), jnp.float32),
        ] + [pltpu.SemaphoreType.DMA] * 2,
    )(_sc_qsum_kernel)


def _sc_gather(nidx2, d2, qdn):
    return _sc_gather_built()(nidx2, d2, qdn)


TA = 64
RE = TA * NB


def _stage2_body(amsij_ref, qdn_ref, emlp_ref, a_ref, edyn_ref, qsum_ref,
                 amsb_ref, nidx_ref, rbf_ref, d_ref, bdyn_ref, blat_ref,
                 wr, br, wb1, bb1, wb2, bb2, wb2r, wm1, bm1, wm2, bm2,
                 aout_o, edn_o, bdn_o, bln_o):
    lane = lax.broadcasted_iota(jnp.int32, (RE, NB), 1)
    row = lax.broadcasted_iota(jnp.int32, (RE, NB), 0)
    sel = (lane == row % NB).astype(jnp.float32)

    def expand_col(x_an):
        z = jnp.broadcast_to(x_an[:, None, :], (TA, NB, NB)).reshape(RE, NB)
        return jnp.sum(z * sel, axis=1, keepdims=True)

    dv = expand_col(d_ref[...])
    x = dv * (1.0 / CUTOFF)
    x2 = x * x
    x4 = x2 * x2
    x8 = x4 * x4
    x9 = x8 * x
    x10 = x9 * x
    x11 = x10 * x
    c1 = (P + 1.0) * (P + 2.0) / 2.0
    c2 = P * (P + 2.0)
    c3 = P * (P + 1.0) / 2.0
    cut = (1.0 - c1 * x9 + c2 * x10 - c3 * x11)
    cut = cut * (dv < CUTOFF).astype(jnp.float32)

    rbfm = (jnp.dot(rbf_ref[...], wr[...]) + br[...]) * cut

    am = amsij_ref[...]
    ai = jnp.broadcast_to(am[:, None, :], (TA, NB, F)).reshape(RE, F)
    ncol = expand_col(nidx_ref[...].astype(jnp.float32))
    vals = lax.broadcasted_iota(jnp.int32, (RE, A), 1)
    oh = (vals == ncol.astype(jnp.int32)).astype(jnp.bfloat16)
    ajv = jnp.dot(oh, amsb_ref[...],
                  preferred_element_type=jnp.float32)
    msij = ai * ajv * rbfm

    h = _silu(jnp.dot(msij, wb1[...]) + bb1[...])
    bij = jnp.dot(h, wb2[...]) + bb2[...]
    h2 = _silu(jnp.dot(msij, wm1[...]) + bm1[...])
    m = jnp.dot(h2, wm2[...]) + bm2[...]

    bdn = bdyn_ref[...] + bij * m
    bdn_o[...] = bdn
    bij_an = (jnp.sum(h.reshape(TA, NB, F) * wb2r[...].reshape(1, 1, F),
                      axis=2) + bb2[...])
    bln_o[...] = blat_ref[...] + bij_an

    dinv = jnp.where(dv > 0.0, 1.0 / dv, 0.0)
    sb = jnp.sum((dinv * bdn).reshape(TA, NB, F), axis=1)
    de = emlp_ref[...] * (qdn_ref[...] * qsum_ref[...] - sb)
    aout_o[...] = a_ref[...] + de
    edn_o[...] = edyn_ref[...] + de


NSTEP = A // TA


@functools.cache
def _stage2_built():
    atom = pl.BlockSpec((TA, F), lambda i: (i, 0))
    atomnb = pl.BlockSpec((TA, NB), lambda i: (i, 0))
    edge = pl.BlockSpec((RE, F), lambda i: (i, 0))
    table = pl.BlockSpec((A, F), lambda i: (i // NSTEP, 0))
    edger = pl.BlockSpec((RE, R), lambda i: (i, 0))
    wspec = lambda s: pl.BlockSpec(s, lambda i: (0, 0))
    return pl.pallas_call(
        _stage2_body,
        grid=(M // TA,),
        in_specs=[atom, atom, atom, atom, atom, atom,
                  table, atomnb, edger, atomnb, edge, atomnb] +
                 [wspec(s) for s in
                  ((R, F), (1, F), (F, F), (1, F), (F, 1), (1, 1), (1, F),
                   (F, F), (1, F), (F, F), (1, F))],
        out_specs=[atom, atom, edge, atomnb],
        out_shape=[
            jax.ShapeDtypeStruct((M, F), jnp.float32),
            jax.ShapeDtypeStruct((M, F), jnp.float32),
            jax.ShapeDtypeStruct((E, F), jnp.float32),
            jax.ShapeDtypeStruct((M, NB), jnp.float32),
        ],
    )


def _stage2(amsij, qdn, emlp, af, edf, qsum, amsb, n32, rbff, df,
            bdf, blf, prbf, pb, pbm):
    wr, brb = prbf
    args_w = [wr, brb.reshape(1, -1)]
    for (w1, b1, w2, b2) in (pb, pbm):
        args_w += [w1, b1.reshape(1, -1), w2, b2.reshape(1, -1)]
    args_w.insert(6, pb[2].reshape(-1)[None, :])
    return _stage2_built()(amsij, qdn, emlp, af, edf, qsum, amsb, n32,
                           rbff, df, bdf, blf, *args_w)


def kernel(a, q_dynamics, b_dynamics, e_dynamics, q_latent, b_latent,
           rbf, D, N, NM, params):
    af = a.reshape(M, F)
    qdf = q_dynamics.reshape(M, F)
    qlf = q_latent.reshape(M, 1)
    edf = e_dynamics.reshape(M, F)
    df = D.reshape(M, NB)
    bdf = b_dynamics.reshape(E, F)
    blf = b_latent.reshape(M, NB)
    nidx2 = N.reshape(E // CH, CH).astype(jnp.int32)
    d2 = D.reshape(E // CH, CH)
    n32 = N.reshape(M, NB).astype(jnp.int32)

    amsij, amsb, qdn, qln, emlp = _stage1(af, qdf, qlf,
                                          params['a'], params['q'],
                                          params['qm'], params['e'])

    qsum = _sc_gather(nidx2, d2, qdn)
    rbff = rbf.reshape(E, R)
    aout, edn, bdn, bln = _stage2(amsij, qdn, emlp, af, edf,
                                  qsum, amsb, n32, rbff, df, bdf, blf,
                                  params['rbf'], params['b'], params['bm'])

    return (aout.reshape(B, A, F),
            qdn.reshape(B, A, F),
            bdn.reshape(B, A, NB, F),
            edn.reshape(B, A, F),
            qln.reshape(B, A, 1),
            bln.reshape(B, A, NB))
